# baseline (speedup 1.0000x reference)
import jax
import jax.numpy as jnp
from jax.experimental import pallas as pl
from jax.experimental.pallas import tpu as pltpu

B = 256
EMB = 768
NCHUNK = 4
ROWS = 256 // NCHUNK


def _copy_body(x_ref, o_ref, *sems):
    cps = [
        pltpu.make_async_copy(
            x_ref.at[pl.ds(i * ROWS, ROWS)],
            o_ref.at[pl.ds(i * ROWS, ROWS)],
            sems[i],
        )
        for i in range(NCHUNK)
    ]
    for c in cps:
        c.start()
    for c in cps:
        c.wait()


def kernel(x_querry, l, x_block, e_k, e_p):
    xb = pl.pallas_call(
        _copy_body,
        in_specs=[pl.BlockSpec(memory_space=pl.ANY)],
        out_specs=pl.BlockSpec(memory_space=pl.ANY),
        out_shape=jax.ShapeDtypeStruct(x_block.shape, x_block.dtype),
        scratch_shapes=[pltpu.SemaphoreType.DMA] * NCHUNK,
    )(x_block)
    Ek = jnp.zeros((B, 4, EMB), jnp.float32)
    return (Ek, Ek, xb)


# speedup vs baseline: 15.0480x; 15.0480x over previous
import jax
import jax.numpy as jnp
from jax.experimental import pallas as pl
from jax.experimental.pallas import tpu as pltpu

B = 256
EMB = 768
ROWS = 8
GRID = 256 // ROWS


def _copy_body(x_ref, o_ref):
    o_ref[...] = x_ref[...]


def kernel(x_querry, l, x_block, e_k, e_p):
    xb = pl.pallas_call(
        _copy_body,
        grid=(GRID,),
        in_specs=[pl.BlockSpec((ROWS, 197, EMB), lambda i: (i, 0, 0))],
        out_specs=pl.BlockSpec((ROWS, 197, EMB), lambda i: (i, 0, 0)),
        out_shape=jax.ShapeDtypeStruct(x_block.shape, x_block.dtype),
    )(x_block)
    Ek = jnp.zeros((B, 4, EMB), jnp.float32)
    return (Ek, Ek, xb)


# speedup vs baseline: 15.0603x; 1.0008x over previous
import jax
import jax.numpy as jnp
from jax.experimental import pallas as pl
from jax.experimental.pallas import tpu as pltpu

B = 256
EMB = 768
ROWS = 8
GRID = 256 // ROWS
NBUF = 4


def _copy_body(x_ref, o_ref, *scratch):
    bufs = scratch[:NBUF]
    sins = scratch[NBUF:2 * NBUF]
    souts = scratch[2 * NBUF:3 * NBUF]
    ins = [
        pltpu.make_async_copy(
            x_ref.at[pl.ds(i * ROWS, ROWS)], bufs[i % NBUF], sins[i % NBUF])
        for i in range(GRID)
    ]
    outs = [
        pltpu.make_async_copy(
            bufs[i % NBUF], o_ref.at[pl.ds(i * ROWS, ROWS)], souts[i % NBUF])
        for i in range(GRID)
    ]
    for i in range(NBUF):
        ins[i].start()
    for i in range(GRID):
        ins[i].wait()
        outs[i].start()
        if i + NBUF < GRID:
            outs[i].wait()
            ins[i + NBUF].start()
    for i in range(GRID - NBUF, GRID):
        if i >= 0:
            outs[i].wait()


def kernel(x_querry, l, x_block, e_k, e_p):
    xb = pl.pallas_call(
        _copy_body,
        in_specs=[pl.BlockSpec(memory_space=pl.ANY)],
        out_specs=pl.BlockSpec(memory_space=pl.ANY),
        out_shape=jax.ShapeDtypeStruct(x_block.shape, x_block.dtype),
        scratch_shapes=([pltpu.VMEM((ROWS, 197, EMB), jnp.float32)] * NBUF
                        + [pltpu.SemaphoreType.DMA] * (2 * NBUF)),
    )(x_block)
    Ek = jnp.zeros((B, 4, EMB), jnp.float32)
    return (Ek, Ek, xb)


# speedup vs baseline: 35.8908x; 2.3831x over previous
import jax
import jax.numpy as jnp
from jax import lax
from jax.experimental import pallas as pl
from jax.experimental.pallas import tpu as pltpu
from jax.experimental.pallas import tpu_sc as plsc

B = 256
KD = 768
POOL = 100
EPL = 8
EMB = 768
HALF = (EPL // 2) * EMB

_NC = 2
_NS = 16
_NW = _NC * _NS
_BPW = B // _NW


def _router_body(q_ref, nk_ref, idx_ref):
    scores = lax.dot_general(
        q_ref[...], nk_ref[...], (((1,), (1,)), ((), ())),
        preferred_element_type=jnp.float32)
    m = jnp.max(scores, axis=1, keepdims=True)
    ii = lax.broadcasted_iota(jnp.int32, scores.shape, 1)
    idx_ref[...] = jnp.min(jnp.where(scores >= m, ii, POOL), axis=1)


def _route(q, nk):
    return pl.pallas_call(
        _router_body,
        out_shape=jax.ShapeDtypeStruct((B,), jnp.int32),
    )(q, nk)


def _gather_body(tab_ref, idx_ref, outk_ref, outv_ref, idx_v, rows_v, sem):
    wid = lax.axis_index("s") * _NC + lax.axis_index("c")
    base = wid * _BPW
    pltpu.sync_copy(idx_ref.at[pl.ds(base, _BPW)], idx_v)
    pltpu.async_copy(tab_ref.at[idx_v], rows_v, sem).wait()
    pltpu.sync_copy(rows_v.at[:, 0], outk_ref.at[pl.ds(base, _BPW)])
    pltpu.sync_copy(rows_v.at[:, 1], outv_ref.at[pl.ds(base, _BPW)])


def _gather(tab, idx):
    mesh = plsc.VectorSubcoreMesh(core_axis_name="c", subcore_axis_name="s")
    f = pl.kernel(
        _gather_body,
        mesh=mesh,
        out_type=[jax.ShapeDtypeStruct((B, HALF), jnp.float32),
                  jax.ShapeDtypeStruct((B, HALF), jnp.float32)],
        scratch_types=[pltpu.VMEM((_BPW,), jnp.int32),
                       pltpu.VMEM((_BPW, 2, HALF), jnp.float32),
                       pltpu.SemaphoreType.DMA],
    )
    return f(tab, idx)


def kernel(x_querry, l, x_block, e_k, e_p):
    n_k = e_k / jnp.maximum(jnp.linalg.norm(e_k, axis=1, keepdims=True), 1e-12)
    q = x_querry / jnp.maximum(
        jnp.linalg.norm(x_querry, axis=1, keepdims=True), 1e-12)
    idx = _route(q, n_k)
    x_block_b, idx = lax.optimization_barrier((x_block, idx))
    tab = e_p.reshape(POOL, 2, HALF)
    ek_half, ev_half = _gather(tab, idx)
    Ek = ek_half.reshape(B, EPL // 2, EMB)
    Ev = ev_half.reshape(B, EPL // 2, EMB)
    return (Ek, Ev, x_block_b)


# speedup vs baseline: 39.2886x; 1.0947x over previous
import jax
import jax.numpy as jnp
from jax import lax
from jax.experimental import pallas as pl
from jax.experimental.pallas import tpu as pltpu
from jax.experimental.pallas import tpu_sc as plsc

B = 256
KD = 768
POOL = 100
EPL = 8
EMB = 768

_NC = 2
_NS = 16
_NW = _NC * _NS
_BPW = B // _NW


def _router_body(q_ref, nk_ref, idx_ref):
    scores = lax.dot_general(
        q_ref[...], nk_ref[...], (((1,), (1,)), ((), ())),
        preferred_element_type=jnp.float32)
    m = jnp.max(scores, axis=1, keepdims=True)
    ii = lax.broadcasted_iota(jnp.int32, scores.shape, 1)
    idx_ref[...] = jnp.min(jnp.where(scores >= m, ii, POOL), axis=1)


def _route(q, nk):
    return pl.pallas_call(
        _router_body,
        out_shape=jax.ShapeDtypeStruct((B,), jnp.int32),
    )(q, nk)


def _gather_body(tab_ref, idx_ref, outk_ref, outv_ref, idx_v, rows_v, sem):
    wid = lax.axis_index("s") * _NC + lax.axis_index("c")
    base = wid * _BPW
    pltpu.sync_copy(idx_ref.at[pl.ds(base, _BPW)], idx_v)
    pltpu.async_copy(tab_ref.at[idx_v], rows_v, sem).wait()
    pltpu.sync_copy(rows_v.at[:, pl.ds(0, EPL // 2)],
                    outk_ref.at[pl.ds(base, _BPW)])
    pltpu.sync_copy(rows_v.at[:, pl.ds(EPL // 2, EPL // 2)],
                    outv_ref.at[pl.ds(base, _BPW)])


def _gather(tab, idx):
    mesh = plsc.VectorSubcoreMesh(core_axis_name="c", subcore_axis_name="s")
    f = pl.kernel(
        _gather_body,
        mesh=mesh,
        out_type=[jax.ShapeDtypeStruct((B, EPL // 2, EMB), jnp.float32),
                  jax.ShapeDtypeStruct((B, EPL // 2, EMB), jnp.float32)],
        scratch_types=[pltpu.VMEM((_BPW,), jnp.int32),
                       pltpu.VMEM((_BPW, EPL, EMB), jnp.float32),
                       pltpu.SemaphoreType.DMA],
    )
    return f(tab, idx)


def kernel(x_querry, l, x_block, e_k, e_p):
    n_k = e_k / jnp.maximum(jnp.linalg.norm(e_k, axis=1, keepdims=True), 1e-12)
    q = x_querry / jnp.maximum(
        jnp.linalg.norm(x_querry, axis=1, keepdims=True), 1e-12)
    idx = _route(q, n_k)
    Ek, Ev = _gather(e_p, idx)
    return (Ek, Ev, x_block)


# speedup vs baseline: 39.3252x; 1.0009x over previous
import jax
import jax.numpy as jnp
from jax import lax
from jax.experimental import pallas as pl
from jax.experimental.pallas import tpu as pltpu
from jax.experimental.pallas import tpu_sc as plsc

B = 256
KD = 768
POOL = 100
EPL = 8
EMB = 768

_NC = 2
_NS = 16
_NW = _NC * _NS
_BPW = B // _NW


def _router_body(q_ref, nk_ref, idx_ref):
    scores = lax.dot_general(
        q_ref[...], nk_ref[...], (((1,), (1,)), ((), ())),
        preferred_element_type=jnp.float32)
    m = jnp.max(scores, axis=1, keepdims=True)
    ii = lax.broadcasted_iota(jnp.int32, scores.shape, 1)
    idx_ref[...] = jnp.min(jnp.where(scores >= m, ii, POOL), axis=1)


def _route(q, nk):
    return pl.pallas_call(
        _router_body,
        out_shape=jax.ShapeDtypeStruct((B,), jnp.int32),
    )(q, nk)


def _gather_body(tab_ref, idx_ref, outk_ref, outv_ref, idx_v, rows_v,
                 sem, semk, semv):
    wid = lax.axis_index("s") * _NC + lax.axis_index("c")
    base = wid * _BPW
    pltpu.sync_copy(idx_ref.at[pl.ds(base, _BPW)], idx_v)
    pltpu.async_copy(tab_ref.at[idx_v], rows_v, sem).wait()
    ck = pltpu.make_async_copy(rows_v.at[:, pl.ds(0, EPL // 2)],
                               outk_ref.at[pl.ds(base, _BPW)], semk)
    cv = pltpu.make_async_copy(rows_v.at[:, pl.ds(EPL // 2, EPL // 2)],
                               outv_ref.at[pl.ds(base, _BPW)], semv)
    ck.start()
    cv.start()
    ck.wait()
    cv.wait()


def _gather(tab, idx):
    mesh = plsc.VectorSubcoreMesh(core_axis_name="c", subcore_axis_name="s")
    f = pl.kernel(
        _gather_body,
        mesh=mesh,
        out_type=[jax.ShapeDtypeStruct((B, EPL // 2, EMB), jnp.float32),
                  jax.ShapeDtypeStruct((B, EPL // 2, EMB), jnp.float32)],
        scratch_types=[pltpu.VMEM((_BPW,), jnp.int32),
                       pltpu.VMEM((_BPW, EPL, EMB), jnp.float32),
                       pltpu.SemaphoreType.DMA,
                       pltpu.SemaphoreType.DMA,
                       pltpu.SemaphoreType.DMA],
    )
    return f(tab, idx)


def kernel(x_querry, l, x_block, e_k, e_p):
    n_k = e_k / jnp.maximum(jnp.linalg.norm(e_k, axis=1, keepdims=True), 1e-12)
    q = x_querry / jnp.maximum(
        jnp.linalg.norm(x_querry, axis=1, keepdims=True), 1e-12)
    idx = _route(q, n_k)
    Ek, Ev = _gather(e_p, idx)
    return (Ek, Ev, x_block)
